# Initial kernel scaffold; baseline (speedup 1.0000x reference)
import jax
import jax.numpy as jnp
from jax.experimental import pallas as pl


def kernel(images, rois, row_ids):
    raise NotImplementedError("write your pallas kernel here")


# speedup vs baseline: 1.3112x; 1.3112x over previous
import dataclasses
import functools

import jax
import jax.numpy as jnp
from jax import lax
from jax.experimental import pallas as pl
from jax.experimental.pallas import tpu as pltpu
from jax.experimental.pallas import tpu_sc as plsc

_POOL = 7
_NPOS = _POOL * _POOL
_H = 512
_W = 512
_C = 128
_NIMG = 8
_NROI = 4000
_NC = 2
_NS = 16
_NW = _NC * _NS
_RPW = _NROI // _NW
_GRP_BASE = (0, 16, 32, 33)
_F1 = jnp.float32(1.0)
_F0 = jnp.float32(0.0)


def _splat_i32(v):
    return jnp.full((16,), v, jnp.int32)


def _roi_pool_body(img_ref, rois_ref, rid_ref, out_ref,
                   rois_v, rid_v, idx_v, w_v, taps_v, out_v,
                   sem_in, sem_out, sem_arg):
    cid = lax.axis_index("c")
    sid = lax.axis_index("s")
    wid = sid * _NC + cid

    pltpu.async_copy(rois_ref.at[wid], rois_v, sem_arg).wait()
    pltpu.async_copy(rid_ref.at[wid], rid_v, sem_arg).wait()

    lanes = lax.iota(jnp.int32, 16)

    def compute_indices(r):
        r4 = _splat_i32(4) * jnp.full((16,), r, jnp.int32)
        x_c = plsc.load_gather(rois_v, [r4])
        y_c = plsc.load_gather(rois_v, [r4 + 1])
        w_s = plsc.load_gather(rois_v, [r4 + 2])
        h_s = plsc.load_gather(rois_v, [r4 + 3])
        b = plsc.load_gather(rid_v, [jnp.full((16,), r, jnp.int32)])
        y1 = y_c - h_s * jnp.float32(0.5)
        y2 = y_c + h_s * jnp.float32(0.5)
        x1 = x_c - w_s * jnp.float32(0.5)
        x2 = x_c + w_s * jnp.float32(0.5)
        ay = y1 * jnp.float32(_H - 1)
        by = (y2 - y1) * jnp.float32(_H - 1)
        ax = x1 * jnp.float32(_W - 1)
        bx = (x2 - x1) * jnp.float32(_W - 1)
        base_b = b * jnp.int32(_H * _W)
        for pb in _GRP_BASE:
            p = lanes + jnp.int32(pb)
            i = p // jnp.int32(_POOL)
            j = p - i * jnp.int32(_POOL)
            cy = i.astype(jnp.float32) / jnp.float32(_POOL - 1)
            cx = j.astype(jnp.float32) / jnp.float32(_POOL - 1)
            ys = ay + cy * by
            xs = ax + cx * bx
            ty = ys.astype(jnp.int32)
            tyf = ty.astype(jnp.float32)
            ya = tyf > ys
            y0f = jnp.where(ya, tyf - _F1, tyf)
            y0 = jnp.where(ya, ty - 1, ty)
            wy = ys - y0f
            tx = xs.astype(jnp.int32)
            txf = tx.astype(jnp.float32)
            xa = txf > xs
            x0f = jnp.where(xa, txf - _F1, txf)
            x0 = jnp.where(xa, tx - 1, tx)
            wx = xs - x0f
            y0c = jnp.minimum(jnp.maximum(y0, 0), jnp.int32(_H - 1))
            y1c = jnp.minimum(jnp.maximum(y0 + 1, 0), jnp.int32(_H - 1))
            x0c = jnp.minimum(jnp.maximum(x0, 0), jnp.int32(_W - 1))
            x1c = jnp.minimum(jnp.maximum(x0 + 1, 0), jnp.int32(_W - 1))
            valid = ((ys >= _F0) & (ys <= jnp.float32(_H - 1))
                     & (xs >= _F0) & (xs <= jnp.float32(_W - 1)))
            m = jnp.where(valid, _F1, _F0)
            omwy = _F1 - wy
            omwx = _F1 - wx
            row0 = base_b + y0c * jnp.int32(_W)
            row1 = base_b + y1c * jnp.int32(_W)
            idx_v[pl.ds(0 * 64 + pb, 16)] = row0 + x0c
            idx_v[pl.ds(1 * 64 + pb, 16)] = row0 + x1c
            idx_v[pl.ds(2 * 64 + pb, 16)] = row1 + x0c
            idx_v[pl.ds(3 * 64 + pb, 16)] = row1 + x1c
            w_v[pl.ds(0 * 64 + pb, 16)] = m * (omwy * omwx)
            w_v[pl.ds(1 * 64 + pb, 16)] = m * (omwy * wx)
            w_v[pl.ds(2 * 64 + pb, 16)] = m * (wy * omwx)
            w_v[pl.ds(3 * 64 + pb, 16)] = m * (wy * wx)

    @pl.loop(0, _RPW)
    def _(r):
        compute_indices(r)
        cps = [pltpu.async_copy(img_ref.at[idx_v.at[pl.ds(t * 64, _NPOS)]],
                                taps_v.at[pl.ds(t * 56, _NPOS)], sem_in)
               for t in range(4)]
        for cp in cps:
            cp.wait()

        cvecs = [lanes + jnp.int32(c * 16) for c in range(_C // 16)]

        @pl.loop(0, _NPOS)
        def _(p):
            psp = jnp.full((16,), p, jnp.int32)
            ws = [plsc.load_gather(w_v, [psp + jnp.int32(t * 64)])
                  for t in range(4)]
            rows = [psp + jnp.int32(t * 56) for t in range(4)]
            for c in range(_C // 16):
                acc = plsc.load_gather(taps_v, [rows[0], cvecs[c]]) * ws[0]
                acc = acc + plsc.load_gather(taps_v, [rows[1], cvecs[c]]) * ws[1]
                acc = acc + plsc.load_gather(taps_v, [rows[2], cvecs[c]]) * ws[2]
                acc = acc + plsc.load_gather(taps_v, [rows[3], cvecs[c]]) * ws[3]
                plsc.store_scatter(out_v, [psp, cvecs[c]], acc)

        g = wid * jnp.int32(_RPW) + r
        pltpu.async_copy(out_v.at[pl.ds(0, _NPOS)], out_ref.at[g], sem_out).wait()


_CP = pltpu.CompilerParams()
if "needs_layout_passes" in pltpu.CompilerParams.__dataclass_fields__:
    _CP = dataclasses.replace(_CP, needs_layout_passes=False)


@functools.partial(
    pl.kernel,
    compiler_params=_CP,
    out_type=jax.ShapeDtypeStruct((_NROI, _NPOS, _C), jnp.float32),
    mesh=plsc.VectorSubcoreMesh(core_axis_name="c", subcore_axis_name="s",
                                num_cores=_NC, num_subcores=_NS),
    scratch_types=[
        pltpu.VMEM((512,), jnp.float32),
        pltpu.VMEM((128,), jnp.int32),
        pltpu.VMEM((256,), jnp.int32),
        pltpu.VMEM((256,), jnp.float32),
        pltpu.VMEM((4 * 56, _C), jnp.float32),
        pltpu.VMEM((56, _C), jnp.float32),
        pltpu.SemaphoreType.DMA,
        pltpu.SemaphoreType.DMA,
        pltpu.SemaphoreType.DMA,
    ],
)
def _roi_pool_sc(img_ref, rois_ref, rid_ref, out_ref, *scratch):
    _roi_pool_body(img_ref, rois_ref, rid_ref, out_ref, *scratch)


def kernel(images, rois, row_ids):
    img = images.reshape(_NIMG * _H * _W, _C)
    rois_g = rois.astype(jnp.float32).reshape(_NW, _RPW, 4)
    rois_p = jnp.zeros((_NW, 128, 4), jnp.float32).at[:, :_RPW].set(rois_g)
    rois_p = rois_p.reshape(_NW, 512)
    rid_p = jnp.zeros((_NW, 128), jnp.int32).at[:, :_RPW].set(
        row_ids.astype(jnp.int32).reshape(_NW, _RPW))
    out = _roi_pool_sc(img, rois_p, rid_p)
    return out.reshape(_NROI, _POOL, _POOL, _C)


# speedup vs baseline: 1.9750x; 1.5063x over previous
import dataclasses
import functools

import jax
import jax.numpy as jnp
import numpy as np
from jax import lax
from jax.experimental import pallas as pl
from jax.experimental.pallas import tpu as pltpu
from jax.experimental.pallas import tpu_sc as plsc

_POOL = 7
_NPOS = _POOL * _POOL
_NTAP = 4 * _NPOS
_H = 512
_W = 512
_C = 128
_NIMG = 8
_NROI = 4000
_NC = 2
_NS = 16
_NW = _NC * _NS
_RPW = _NROI // _NW
_GRP_BASE = (0, 16, 32, 33)
_ISLOT = 256
_TSLOT = 224
_OSLOT = 56
_F1 = jnp.float32(1.0)
_F0 = jnp.float32(0.0)

_P = np.arange(_NPOS)
_AUXF = np.zeros(128, np.float32)
_AUXF[:_NPOS] = (_P // _POOL).astype(np.float32) / np.float32(_POOL - 1)
_AUXF[64:64 + _NPOS] = (_P % _POOL).astype(np.float32) / np.float32(_POOL - 1)
_AUXI = np.arange(128, dtype=np.int32)


def _roi_pool_body(img_ref, rois_ref, rid_ref, auxf_ref, auxi_ref, out_ref,
                   rois_v, rid_v, auxf_v, auxi_v, idx_v, w_v, taps_v, out_v,
                   sem_in, sem_out, sem_arg):
    cid = lax.axis_index("c")
    sid = lax.axis_index("s")
    wid = sid * _NC + cid

    pltpu.async_copy(rois_ref.at[wid], rois_v, sem_arg).wait()
    pltpu.async_copy(rid_ref.at[wid], rid_v, sem_arg).wait()
    pltpu.async_copy(auxf_ref, auxf_v, sem_arg).wait()
    pltpu.async_copy(auxi_ref, auxi_v, sem_arg).wait()

    cvecs = [auxi_v[pl.ds(c * 16, 16)] for c in range(_C // 16)]

    def compute_indices(r, s):
        r4 = jnp.full((16,), r * 4, jnp.int32)
        x_c = plsc.load_gather(rois_v, [r4])
        y_c = plsc.load_gather(rois_v, [r4 + 1])
        w_s = plsc.load_gather(rois_v, [r4 + 2])
        h_s = plsc.load_gather(rois_v, [r4 + 3])
        b = plsc.load_gather(rid_v, [jnp.full((16,), r, jnp.int32)])
        y1 = y_c - h_s * jnp.float32(0.5)
        y2 = y_c + h_s * jnp.float32(0.5)
        x1 = x_c - w_s * jnp.float32(0.5)
        x2 = x_c + w_s * jnp.float32(0.5)
        ay = y1 * jnp.float32(_H - 1)
        by = (y2 - y1) * jnp.float32(_H - 1)
        ax = x1 * jnp.float32(_W - 1)
        bx = (x2 - x1) * jnp.float32(_W - 1)
        base_b = b * jnp.int32(_H * _W)
        for pb in _GRP_BASE:
            cy = auxf_v[pl.ds(pb, 16)]
            cx = auxf_v[pl.ds(64 + pb, 16)]
            ys = ay + cy * by
            xs = ax + cx * bx
            ty = ys.astype(jnp.int32)
            tyf = ty.astype(jnp.float32)
            ya = tyf > ys
            y0f = jnp.where(ya, tyf - _F1, tyf)
            y0 = jnp.where(ya, ty - 1, ty)
            wy = ys - y0f
            tx = xs.astype(jnp.int32)
            txf = tx.astype(jnp.float32)
            xa = txf > xs
            x0f = jnp.where(xa, txf - _F1, txf)
            x0 = jnp.where(xa, tx - 1, tx)
            wx = xs - x0f
            y0c = jnp.minimum(jnp.maximum(y0, 0), jnp.int32(_H - 1))
            y1c = jnp.minimum(jnp.maximum(y0 + 1, 0), jnp.int32(_H - 1))
            x0c = jnp.minimum(jnp.maximum(x0, 0), jnp.int32(_W - 1))
            x1c = jnp.minimum(jnp.maximum(x0 + 1, 0), jnp.int32(_W - 1))
            valid = ((ys >= _F0) & (ys <= jnp.float32(_H - 1))
                     & (xs >= _F0) & (xs <= jnp.float32(_W - 1)))
            m = jnp.where(valid, _F1, _F0)
            omwy = _F1 - wy
            omwx = _F1 - wx
            row0 = base_b + y0c * jnp.int32(_W)
            row1 = base_b + y1c * jnp.int32(_W)
            taps = (row0 + x0c, row0 + x1c, row1 + x0c, row1 + x1c)
            wts = (m * (omwy * omwx), m * (omwy * wx),
                   m * (wy * omwx), m * (wy * wx))
            for t in range(4):
                idx_v[pl.ds(s * _ISLOT + t * _NPOS + pb, 16)] = taps[t]
                w_v[pl.ds(s * _ISLOT + t * 64 + pb, 16)] = wts[t]

    def gather_slices(s):
        return (img_ref.at[idx_v.at[pl.ds(s * _ISLOT, _NTAP)]],
                taps_v.at[pl.ds(s * _TSLOT, _NTAP)])

    def fire_gather(r, s):
        compute_indices(r, s)
        src, dst = gather_slices(s)
        pltpu.async_copy(src, dst, sem_in[s])

    def wait_gather(s):
        src, dst = gather_slices(s)
        pltpu.make_async_copy(src, dst, sem_in[s]).wait()

    def wait_out(s):
        pltpu.make_async_copy(out_v.at[pl.ds(s * _OSLOT, _NPOS)],
                              out_ref.at[0], sem_out[s]).wait()

    def consume(r, s):
        @pl.loop(0, _NPOS, unroll=7)
        def _(p):
            psp = jnp.full((16,), p, jnp.int32)
            ws = [plsc.load_gather(w_v, [psp + jnp.int32(s * _ISLOT + t * 64)])
                  for t in range(4)]
            rows = [psp + jnp.int32(s * _TSLOT + t * _NPOS) for t in range(4)]
            for c in range(_C // 16):
                acc = plsc.load_gather(taps_v, [rows[0], cvecs[c]]) * ws[0]
                acc = acc + plsc.load_gather(taps_v, [rows[1], cvecs[c]]) * ws[1]
                acc = acc + plsc.load_gather(taps_v, [rows[2], cvecs[c]]) * ws[2]
                acc = acc + plsc.load_gather(taps_v, [rows[3], cvecs[c]]) * ws[3]
                plsc.store_scatter(out_v, [psp + jnp.int32(s * _OSLOT), cvecs[c]],
                                   acc)

        g = wid * jnp.int32(_RPW) + r
        pltpu.async_copy(out_v.at[pl.ds(s * _OSLOT, _NPOS)], out_ref.at[g],
                         sem_out[s])

    @pl.loop(0, _RPW // 2 + 2)
    def _(k):
        r = 2 * k - 2

        @pl.when(k > 0)
        def _():
            wait_gather(0)

            @pl.when(k > 1)
            def _():
                wait_out(0)
            consume(r, 0)

        @pl.when(r + 2 < _RPW)
        def _():
            fire_gather(r + 2, 0)

        @pl.when((k > 0) & (r + 1 < _RPW))
        def _():
            wait_gather(1)

            @pl.when(k > 1)
            def _():
                wait_out(1)
            consume(r + 1, 1)

        @pl.when(r + 3 < _RPW)
        def _():
            fire_gather(r + 3, 1)

    wait_out(1)
    wait_out(0)


_CP = pltpu.CompilerParams()
if "needs_layout_passes" in pltpu.CompilerParams.__dataclass_fields__:
    _CP = dataclasses.replace(_CP, needs_layout_passes=False)


@functools.partial(
    pl.kernel,
    compiler_params=_CP,
    out_type=jax.ShapeDtypeStruct((_NROI, _NPOS, _C), jnp.float32),
    mesh=plsc.VectorSubcoreMesh(core_axis_name="c", subcore_axis_name="s",
                                num_cores=_NC, num_subcores=_NS),
    scratch_types=[
        pltpu.VMEM((512,), jnp.float32),
        pltpu.VMEM((128,), jnp.int32),
        pltpu.VMEM((128,), jnp.float32),
        pltpu.VMEM((128,), jnp.int32),
        pltpu.VMEM((2 * _ISLOT,), jnp.int32),
        pltpu.VMEM((2 * _ISLOT,), jnp.float32),
        pltpu.VMEM((2 * _TSLOT, _C), jnp.float32),
        pltpu.VMEM((2 * _OSLOT, _C), jnp.float32),
        [pltpu.SemaphoreType.DMA, pltpu.SemaphoreType.DMA],
        [pltpu.SemaphoreType.DMA, pltpu.SemaphoreType.DMA],
        pltpu.SemaphoreType.DMA,
    ],
)
def _roi_pool_sc(img_ref, rois_ref, rid_ref, auxf_ref, auxi_ref, out_ref,
                 *scratch):
    _roi_pool_body(img_ref, rois_ref, rid_ref, auxf_ref, auxi_ref, out_ref,
                   *scratch)


def kernel(images, rois, row_ids):
    img = images.reshape(_NIMG * _H * _W, _C)
    rois_g = rois.astype(jnp.float32).reshape(_NW, _RPW, 4)
    rois_p = jnp.zeros((_NW, 128, 4), jnp.float32).at[:, :_RPW].set(rois_g)
    rois_p = rois_p.reshape(_NW, 512)
    rid_p = jnp.zeros((_NW, 128), jnp.int32).at[:, :_RPW].set(
        row_ids.astype(jnp.int32).reshape(_NW, _RPW))
    auxf = jnp.asarray(_AUXF)
    auxi = jnp.asarray(_AUXI)
    out = _roi_pool_sc(img, rois_p, rid_p, auxf, auxi)
    return out.reshape(_NROI, _POOL, _POOL, _C)
